# Initial kernel scaffold; baseline (speedup 1.0000x reference)
import jax
import jax.numpy as jnp
from jax.experimental import pallas as pl


def kernel(idx, logits_table):
    raise NotImplementedError("write your pallas kernel here")


# speedup vs baseline: 1.0243x; 1.0243x over previous
import functools

import jax
import jax.numpy as jnp
from jax import lax
from jax.experimental import pallas as pl
from jax.experimental.pallas import tpu as pltpu
from jax.experimental.pallas import tpu_sc as plsc

_V = 1000
_D = 1000
_B, _L = 1024, 50
_BTOT = _B * _L

_NC, _NS = 2, 16
_NW = _NC * _NS
_BPW = _BTOT // _NW
_CH = 40
_NCH = _BPW // _CH


def _make_gather():
    mesh = plsc.VectorSubcoreMesh(core_axis_name="c", subcore_axis_name="s")

    @functools.partial(
        pl.kernel,
        out_type=jax.ShapeDtypeStruct((_BTOT, _D), jnp.float32),
        mesh=mesh,
        scratch_types=[
            pltpu.VMEM((_NCH, _CH), jnp.int32),
            pltpu.VMEM((2, _CH, _D), jnp.float32),
            pltpu.SemaphoreType.DMA,
            pltpu.SemaphoreType.DMA,
            pltpu.SemaphoreType.DMA,
            pltpu.SemaphoreType.DMA,
        ],
        compiler_params=pltpu.CompilerParams(use_tc_tiling_on_sc=False),
    )
    def gather_kernel(idx_hbm, table_hbm, out_hbm, idx_v, buf, g0s, g1s,
                      s0s, s1s):
        wid = lax.axis_index("s") * _NC + lax.axis_index("c")
        base = wid * _BPW
        pltpu.sync_copy(idx_hbm.at[wid], idx_v)

        def pair(i, carry):
            c0 = 2 * i
            c1 = c0 + 1
            g0 = pltpu.async_copy(table_hbm.at[idx_v.at[c0]], buf.at[0], g0s)
            g1 = pltpu.async_copy(table_hbm.at[idx_v.at[c1]], buf.at[1], g1s)
            g0.wait()
            s0 = pltpu.async_copy(
                buf.at[0], out_hbm.at[pl.ds(base + c0 * _CH, _CH)], s0s)
            g1.wait()
            s1 = pltpu.async_copy(
                buf.at[1], out_hbm.at[pl.ds(base + c1 * _CH, _CH)], s1s)
            s0.wait()
            s1.wait()
            return carry

        lax.fori_loop(0, _NCH // 2, pair, 0)

    return gather_kernel


_gather = _make_gather()


@jax.jit
def kernel(idx, logits_table):
    idx_w = idx.reshape(_NW, _NCH, _CH).astype(jnp.int32)
    out = _gather(idx_w, logits_table)
    return out.reshape(_B, _L, _D)


# speedup vs baseline: 1.0308x; 1.0064x over previous
import functools

import jax
import jax.numpy as jnp
from jax import lax
from jax.experimental import pallas as pl
from jax.experimental.pallas import tpu as pltpu
from jax.experimental.pallas import tpu_sc as plsc

_V = 1000
_D = 1000
_B, _L = 1024, 50
_BTOT = _B * _L

_NC, _NS = 2, 16
_NW = _NC * _NS
_BPW = _BTOT // _NW
_NB = 4
_CH = 20
_NCH = _BPW // _CH


def _make_gather():
    mesh = plsc.VectorSubcoreMesh(core_axis_name="c", subcore_axis_name="s")

    @functools.partial(
        pl.kernel,
        out_type=jax.ShapeDtypeStruct((_BTOT, _D), jnp.float32),
        mesh=mesh,
        scratch_types=[
            pltpu.VMEM((_NCH, _CH), jnp.int32),
            pltpu.VMEM((_NB, _CH, _D), jnp.float32),
            [pltpu.SemaphoreType.DMA] * _NB,
            [pltpu.SemaphoreType.DMA] * _NB,
        ],
        compiler_params=pltpu.CompilerParams(use_tc_tiling_on_sc=False),
    )
    def gather_kernel(idx_hbm, table_hbm, out_hbm, idx_v, buf, gsems, ssems):
        wid = lax.axis_index("s") * _NC + lax.axis_index("c")
        base = wid * _BPW
        pltpu.sync_copy(idx_hbm.at[wid], idx_v)

        def g_desc(c, b):
            return pltpu.make_async_copy(
                table_hbm.at[idx_v.at[c]], buf.at[b], gsems[b])

        def s_desc(c, b):
            return pltpu.make_async_copy(
                buf.at[b], out_hbm.at[pl.ds(base + c * _CH, _CH)], ssems[b])

        g_desc(0, 0).start()
        g_desc(1, 1).start()

        def group(g, carry):
            c0 = g * _NB
            for b in range(_NB):
                c = c0 + b
                g_desc(c, b).wait()
                s_desc(c, b).start()
                f = c + 2
                bf = (b + 2) % _NB

                @pl.when(f < _NCH)
                def _():
                    @pl.when(f >= _NB)
                    def _():
                        s_desc(f - _NB, bf).wait()
                    g_desc(f, bf).start()

            return carry

        lax.fori_loop(0, _NCH // _NB, group, 0)
        s_desc(_NCH - 2, (_NCH - 2) % _NB).wait()
        s_desc(_NCH - 1, (_NCH - 1) % _NB).wait()

    return gather_kernel


_gather = _make_gather()


@jax.jit
def kernel(idx, logits_table):
    idx_w = idx.reshape(_NW, _NCH, _CH).astype(jnp.int32)
    out = _gather(idx_w, logits_table)
    return out.reshape(_B, _L, _D)
